# baseline (speedup 1.0000x reference)
import functools

import jax
import jax.numpy as jnp
import numpy as np
from jax import lax
from jax.experimental import pallas as pl
from jax.experimental.pallas import tpu as pltpu

B, L, H, K, VOCAB = 4, 512, 256, 16, 4
N_ENC, N_DEC = 3, 3
N_RBF, N_POS = 16, 16
NODE_IN = 9
EDGE_IN = N_RBF + N_POS

N = B * L
E = B * L * K
NLAYERS = N_ENC + N_DEC
EBLK = 2048
NBLK = EBLK // K
N_EBLKS = E // EBLK
BLKS_PER_BATCH = (L * K) // EBLK
f32 = jnp.float32
bf16 = jnp.bfloat16


def _ln(x, g, b):
    mu = jnp.mean(x, -1, keepdims=True)
    var = jnp.var(x, -1, keepdims=True)
    return (x - mu) / jnp.sqrt(var + 1e-5) * g + b


def _eemb_body(eraw_ref, ew_ref, eb_ref, g_ref, b_ref, he_ref):
    h = jnp.dot(eraw_ref[...], ew_ref[...], preferred_element_type=f32)
    he_ref[...] = _ln(h + eb_ref[...], g_ref[...], b_ref[...]).astype(bf16)


def _mega_body(nraw_ref, soh_ref, mask_ref, src_ref, he_hbm,
               nw_ref, nb_ref, ws_ref, nlg_ref, nlb_ref,
               w1e_ref, w1s_ref, w1d_ref, b1_ref,
               w2_ref, b2_ref, w3_ref, b3_ref,
               g1_ref, bb1_ref, f1_ref, fb1_ref, f2_ref, fb2_ref,
               g2_ref, bb2_ref, wsk_ref, bsk_ref, gsk_ref, bsk2_ref,
               wr_ref, br_ref, p1_ref, p2_ref, p2b_ref,
               logits_ref, prj_ref,
               hv_s, a_s, bd_s, agg_s, he_buf, sem):
    h = jnp.dot(nraw_ref[...], nw_ref[...], preferred_element_type=f32)
    h = h + nb_ref[...] + jnp.dot(soh_ref[...], ws_ref[...],
                                  preferred_element_type=f32)
    hv0 = _ln(h, nlg_ref[...], nlb_ref[...]) * mask_ref[...]
    hv_s[...] = hv0
    a_s[...] = jnp.dot(hv0, w1s_ref[0], preferred_element_type=f32)
    bd_s[...] = jnp.dot(hv0, w1d_ref[0], preferred_element_type=f32)

    def he_copy(blk, slot):
        return pltpu.make_async_copy(
            he_hbm.at[pl.ds(blk * EBLK, EBLK), :], he_buf.at[slot],
            sem.at[slot])

    def layer_body(l, carry):
        w1e_b = w1e_ref[l].astype(bf16)
        b1v = b1_ref[l]
        w2v = w2_ref[l].astype(bf16)
        b2v = b2_ref[l]

        he_copy(0, 0).start()

        def edge_body(blk, c):
            slot = lax.rem(blk, 2)
            he_copy(blk, slot).wait()

            @pl.when(blk + 1 < N_EBLKS)
            def _():
                he_copy(blk + 1, lax.rem(blk + 1, 2)).start()

            base = (blk // BLKS_PER_BATCH) * L
            src_local = (src_ref[blk, 0, :] - base).astype(jnp.int16)
            oh = jnp.where(
                src_local[:, None] ==
                lax.broadcasted_iota(jnp.int16, (EBLK, L), 1),
                bf16(1.0), bf16(0.0))
            a_blk = a_s[pl.ds(base, L), :].astype(bf16)
            gath = jnp.dot(oh, a_blk, preferred_element_type=f32)
            epart = jnp.dot(he_buf[slot], w1e_b,
                            preferred_element_type=f32)
            pre = (epart + b1v + gath).reshape(NBLK, K, H) \
                + bd_s[pl.ds(blk * NBLK, NBLK), :][:, None, :]
            m = jax.nn.gelu(pre.astype(bf16)).reshape(EBLK, H)
            m2 = jnp.dot(m, w2v, preferred_element_type=f32) + b2v
            m2 = jax.nn.gelu(m2.astype(bf16)).astype(f32)
            agg_s[pl.ds(blk * NBLK, NBLK), :] = \
                jnp.sum(m2.reshape(NBLK, K, H), axis=1)
            return c

        lax.fori_loop(0, N_EBLKS, edge_body, 0)

        bdot = lambda x, w: jnp.dot(x.astype(bf16), w.astype(bf16),
                                    preferred_element_type=f32)
        hv = hv_s[...]
        agg = bdot(agg_s[...] * (1.0 / K), w3_ref[l]) + b3_ref[l]
        hh = _ln(hv + agg, g1_ref[l], bb1_ref[l])
        ff = bdot(jnp.maximum(bdot(hh, f1_ref[l]) + fb1_ref[l], 0.0),
                  f2_ref[l]) + fb2_ref[l]
        hh = _ln(hh + ff, g2_ref[l], bb2_ref[l])
        sk = jnp.maximum(bdot(hh, wsk_ref[l]) + bsk_ref[l], 0.0)
        hv_new = hv + _ln(sk, gsk_ref[l], bsk2_ref[l])
        hv_s[...] = hv_new
        ln = jnp.minimum(l + 1, NLAYERS - 1)
        a_s[...] = jnp.dot(hv_new, w1s_ref[ln], preferred_element_type=f32)
        bd_s[...] = jnp.dot(hv_new, w1d_ref[ln], preferred_element_type=f32)
        return carry

    lax.fori_loop(0, NLAYERS, layer_body, 0)

    hv = hv_s[...]
    logits_ref[...] = jnp.dot(hv, wr_ref[...],
                              preferred_element_type=f32) + br_ref[...]
    ge = jnp.mean(hv.reshape(B, L, H), axis=1)
    prj = jnp.maximum(jnp.dot(ge, p1_ref[...],
                              preferred_element_type=f32), 0.0)
    prj_ref[...] = jnp.dot(prj, p2_ref[...],
                           preferred_element_type=f32) + p2b_ref[...]


def _full(shape):
    return pl.BlockSpec(shape, lambda *_: tuple(0 for _ in shape))


def kernel(X, S, mask, params):
    p = params

    center = X[:, :, 1, :]
    diff = center[:, :, None, :] - center[:, None, :, :]
    D = jnp.sqrt(jnp.sum(diff * diff, -1) + 1e-8)
    D = D + jnp.eye(L, dtype=f32)[None] * 1e6
    negD, nbr = jax.lax.top_k(-D, K)
    d_nbr = -negD
    centers = jnp.linspace(2.0, 22.0, N_RBF)
    sigma = (22.0 - 2.0) / N_RBF
    rbf = jnp.exp(-(((d_nbr[..., None] - centers) / sigma) ** 2))
    rel = (nbr - jnp.arange(L)[None, :, None]).astype(f32)
    freq = jnp.exp(-jnp.arange(N_POS // 2, dtype=f32)
                   * (np.log(10000.0) / (N_POS // 2)))
    ang = rel[..., None] * freq
    posenc = jnp.concatenate([jnp.sin(ang), jnp.cos(ang)], -1)
    e_raw = jnp.concatenate([rbf, posenc], -1).reshape(E, EDGE_IN)

    def unit(v):
        return v / (jnp.linalg.norm(v, axis=-1, keepdims=True) + 1e-8)
    v1 = unit(X[:, :, 1] - X[:, :, 0])
    v2 = unit(X[:, :, 2] - X[:, :, 1])
    v3 = unit(jnp.roll(center, -1, axis=1) - center)
    node_raw = jnp.concatenate([v1, v2, v3], -1).reshape(N, NODE_IN)

    offs = (jnp.arange(B, dtype=jnp.int32) * L)[:, None, None]
    src = (nbr.astype(jnp.int32) + offs).reshape(N_EBLKS, 1, EBLK)
    s_oh = jax.nn.one_hot(S.reshape(N), VOCAB, dtype=f32)
    mask_col = mask.reshape(N, 1)

    row = lambda v: v.reshape(1, -1)
    layers = ['enc%d' % i for i in range(N_ENC)] + \
             ['dec%d' % i for i in range(N_DEC)]
    stk = lambda name: jnp.stack([p[pre + name] for pre in layers])
    stkr = lambda name: jnp.stack([row(p[pre + name]) for pre in layers])

    he = pl.pallas_call(
        _eemb_body,
        grid=(8,),
        in_specs=[pl.BlockSpec((E // 8, EDGE_IN), lambda i: (i, 0)),
                  _full((EDGE_IN, H)), _full((1, H)),
                  _full((1, H)), _full((1, H))],
        out_specs=pl.BlockSpec((E // 8, H), lambda i: (i, 0)),
        out_shape=jax.ShapeDtypeStruct((E, H), bf16),
        compiler_params=pltpu.CompilerParams(
            dimension_semantics=("parallel",)),
    )(e_raw, p['edge_W'], row(p['edge_b']),
      row(p['edge_ln_g']), row(p['edge_ln_b']))

    logits, prjs = pl.pallas_call(
        _mega_body,
        grid=(),
        in_specs=[_full((N, NODE_IN)), _full((N, VOCAB)), _full((N, 1)),
                  _full((N_EBLKS, 1, EBLK)),
                  pl.BlockSpec(memory_space=pl.ANY),
                  _full((NODE_IN, H)), _full((1, H)), _full((VOCAB, H)),
                  _full((1, H)), _full((1, H)),
                  _full((NLAYERS, H, H)), _full((NLAYERS, H, H)),
                  _full((NLAYERS, H, H)), _full((NLAYERS, 1, H)),
                  _full((NLAYERS, H, H)), _full((NLAYERS, 1, H)),
                  _full((NLAYERS, H, H)), _full((NLAYERS, 1, H)),
                  _full((NLAYERS, 1, H)), _full((NLAYERS, 1, H)),
                  _full((NLAYERS, H, 4 * H)), _full((NLAYERS, 1, 4 * H)),
                  _full((NLAYERS, 4 * H, H)), _full((NLAYERS, 1, H)),
                  _full((NLAYERS, 1, H)), _full((NLAYERS, 1, H)),
                  _full((NLAYERS, H, H)), _full((NLAYERS, 1, H)),
                  _full((NLAYERS, 1, H)), _full((NLAYERS, 1, H)),
                  _full((H, VOCAB)), _full((1, VOCAB)),
                  _full((H, H)), _full((H, H)), _full((1, H))],
        out_specs=[_full((N, VOCAB)), _full((B, H))],
        out_shape=[jax.ShapeDtypeStruct((N, VOCAB), f32),
                   jax.ShapeDtypeStruct((B, H), f32)],
        scratch_shapes=[pltpu.VMEM((N, H), f32), pltpu.VMEM((N, H), f32),
                        pltpu.VMEM((N, H), f32), pltpu.VMEM((N, H), f32),
                        pltpu.VMEM((2, EBLK, H), bf16),
                        pltpu.SemaphoreType.DMA((2,))],
    )(node_raw, s_oh, mask_col, src, he,
      p['node_W'], row(p['node_b']), p['W_s'],
      row(p['node_ln_g']), row(p['node_ln_b']),
      stk('_m1_W')[:, :H], stk('_m1_W')[:, H:2 * H], stk('_m1_W')[:, 2 * H:],
      stkr('_m1_b'),
      stk('_m2_W'), stkr('_m2_b'), stk('_m3_W'), stkr('_m3_b'),
      stkr('_ln1_g'), stkr('_ln1_b'),
      stk('_f1_W'), stkr('_f1_b'), stk('_f2_W'), stkr('_f2_b'),
      stkr('_ln2_g'), stkr('_ln2_b'),
      stk('_skip_W'), stkr('_skip_b'), stkr('_skln_g'), stkr('_skln_b'),
      p['readout_W'], row(p['readout_b']),
      p['proj1_W'], p['proj2_W'], row(p['proj2_b']))

    return logits, S.reshape(-1), prjs


# speedup vs baseline: 1.0093x; 1.0093x over previous
import functools

import jax
import jax.numpy as jnp
import numpy as np
from jax import lax
from jax.experimental import pallas as pl
from jax.experimental.pallas import tpu as pltpu

B, L, H, K, VOCAB = 4, 512, 256, 16, 4
N_ENC, N_DEC = 3, 3
N_RBF, N_POS = 16, 16
NODE_IN = 9
EDGE_IN = N_RBF + N_POS

N = B * L
E = B * L * K
NLAYERS = N_ENC + N_DEC
EBLK = 2048
NBLK = EBLK // K
N_EBLKS = E // EBLK
BLKS_PER_BATCH = (L * K) // EBLK
f32 = jnp.float32
bf16 = jnp.bfloat16


def _ln(x, g, b):
    mu = jnp.mean(x, -1, keepdims=True)
    var = jnp.var(x, -1, keepdims=True)
    return (x - mu) / jnp.sqrt(var + 1e-5) * g + b


def _eemb_body(eraw_ref, ew_ref, eb_ref, g_ref, b_ref, he_ref):
    h = jnp.dot(eraw_ref[...], ew_ref[...], preferred_element_type=f32)
    he_ref[...] = _ln(h + eb_ref[...], g_ref[...], b_ref[...]).astype(bf16)


def _mega_body(nraw_ref, soh_ref, mask_ref, src_ref, he_hbm,
               nw_ref, nb_ref, ws_ref, nlg_ref, nlb_ref,
               w1e_ref, w1s_ref, w1d_ref, b1_ref,
               w2_ref, b2_ref, w3_ref, b3_ref,
               g1_ref, bb1_ref, f1_ref, fb1_ref, f2_ref, fb2_ref,
               g2_ref, bb2_ref, wsk_ref, bsk_ref, gsk_ref, bsk2_ref,
               wr_ref, br_ref, p1_ref, p2_ref, p2b_ref,
               logits_ref, prj_ref,
               hv_s, a_s, bd_s, agg_s, he_buf, sem):
    h = jnp.dot(nraw_ref[...], nw_ref[...], preferred_element_type=f32)
    h = h + nb_ref[...] + jnp.dot(soh_ref[...], ws_ref[...],
                                  preferred_element_type=f32)
    hv0 = _ln(h, nlg_ref[...], nlb_ref[...]) * mask_ref[...]
    hv_s[...] = hv0
    a_s[...] = jnp.dot(hv0, w1s_ref[0], preferred_element_type=f32)
    bd_s[...] = jnp.dot(hv0, w1d_ref[0], preferred_element_type=f32)

    def he_copy(blk, slot):
        return pltpu.make_async_copy(
            he_hbm.at[pl.ds(blk * EBLK, EBLK), :], he_buf.at[slot],
            sem.at[slot])

    def layer_body(l, carry):
        w1e_b = w1e_ref[l].astype(bf16)
        b1v = b1_ref[l]
        w2v = w2_ref[l].astype(bf16)
        b2v = b2_ref[l]

        he_copy(0, 0).start()

        def edge_body(blk, c):
            slot = lax.rem(blk, 2)
            he_copy(blk, slot).wait()

            @pl.when(blk + 1 < N_EBLKS)
            def _():
                he_copy(blk + 1, lax.rem(blk + 1, 2)).start()

            base = (blk // BLKS_PER_BATCH) * L
            src_local = (src_ref[blk, 0, :] - base).astype(jnp.int16)
            oh = jnp.where(
                src_local[:, None] ==
                lax.broadcasted_iota(jnp.int16, (EBLK, L), 1),
                bf16(1.0), bf16(0.0))
            a_blk = a_s[pl.ds(base, L), :].astype(bf16)
            gath = jnp.dot(oh, a_blk, preferred_element_type=f32)
            epart = jnp.dot(he_buf[slot], w1e_b,
                            preferred_element_type=f32)
            pre = (epart + b1v + gath).reshape(NBLK, K, H) \
                + bd_s[pl.ds(blk * NBLK, NBLK), :][:, None, :]
            m = jax.nn.gelu(pre.astype(bf16)).reshape(EBLK, H)
            m2 = jnp.dot(m, w2v, preferred_element_type=f32) + b2v
            m2 = jax.nn.gelu(m2.astype(bf16)).astype(f32)
            agg_s[pl.ds(blk * NBLK, NBLK), :] = \
                jnp.sum(m2.reshape(NBLK, K, H), axis=1)
            return c

        lax.fori_loop(0, N_EBLKS, edge_body, 0)

        fdot = lambda x, w: jnp.dot(x, w, preferred_element_type=f32)
        hv = hv_s[...]
        agg = fdot(agg_s[...] * (1.0 / K), w3_ref[l]) + b3_ref[l]
        hh = _ln(hv + agg, g1_ref[l], bb1_ref[l])
        ff = fdot(jnp.maximum(fdot(hh, f1_ref[l]) + fb1_ref[l], 0.0),
                  f2_ref[l]) + fb2_ref[l]
        hh = _ln(hh + ff, g2_ref[l], bb2_ref[l])
        sk = jnp.maximum(fdot(hh, wsk_ref[l]) + bsk_ref[l], 0.0)
        hv_new = hv + _ln(sk, gsk_ref[l], bsk2_ref[l])
        hv_s[...] = hv_new
        ln = jnp.minimum(l + 1, NLAYERS - 1)
        a_s[...] = jnp.dot(hv_new, w1s_ref[ln], preferred_element_type=f32)
        bd_s[...] = jnp.dot(hv_new, w1d_ref[ln], preferred_element_type=f32)
        return carry

    lax.fori_loop(0, NLAYERS, layer_body, 0)

    hv = hv_s[...]
    logits_ref[...] = jnp.dot(hv, wr_ref[...],
                              preferred_element_type=f32) + br_ref[...]
    ge = jnp.mean(hv.reshape(B, L, H), axis=1)
    prj = jnp.maximum(jnp.dot(ge, p1_ref[...],
                              preferred_element_type=f32), 0.0)
    prj_ref[...] = jnp.dot(prj, p2_ref[...],
                           preferred_element_type=f32) + p2b_ref[...]


def _full(shape):
    return pl.BlockSpec(shape, lambda *_: tuple(0 for _ in shape))


def kernel(X, S, mask, params):
    p = params

    center = X[:, :, 1, :]
    c0 = center - jnp.mean(center, axis=1, keepdims=True)
    n2 = jnp.sum(c0 * c0, -1)
    G = jnp.einsum('bic,bjc->bij', c0, c0)
    d2 = n2[:, :, None] + n2[:, None, :] - 2.0 * G
    D = jnp.sqrt(jnp.maximum(d2, 0.0) + 1e-8)
    D = D + jnp.eye(L, dtype=f32)[None] * 1e6
    nD4 = (-D).reshape(B, L, 4, 128)
    v1, i1 = jax.lax.top_k(nD4, K)
    v1 = v1.reshape(B, L, 4 * K)
    i1 = (i1 + (jnp.arange(4, dtype=jnp.int32) * 128)[None, None, :, None]
          ).reshape(B, L, 4 * K)
    negD, i2 = jax.lax.top_k(v1, K)
    nbr = jnp.take_along_axis(i1, i2, axis=-1)
    d_nbr = -negD
    centers = jnp.linspace(2.0, 22.0, N_RBF)
    sigma = (22.0 - 2.0) / N_RBF
    rbf = jnp.exp(-(((d_nbr[..., None] - centers) / sigma) ** 2))
    rel = (nbr - jnp.arange(L)[None, :, None]).astype(f32)
    freq = jnp.exp(-jnp.arange(N_POS // 2, dtype=f32)
                   * (np.log(10000.0) / (N_POS // 2)))
    ang = rel[..., None] * freq
    posenc = jnp.concatenate([jnp.sin(ang), jnp.cos(ang)], -1)
    e_raw = jnp.concatenate([rbf, posenc], -1).reshape(E, EDGE_IN)

    def unit(v):
        return v / (jnp.linalg.norm(v, axis=-1, keepdims=True) + 1e-8)
    v1 = unit(X[:, :, 1] - X[:, :, 0])
    v2 = unit(X[:, :, 2] - X[:, :, 1])
    v3 = unit(jnp.roll(center, -1, axis=1) - center)
    node_raw = jnp.concatenate([v1, v2, v3], -1).reshape(N, NODE_IN)

    offs = (jnp.arange(B, dtype=jnp.int32) * L)[:, None, None]
    src = (nbr.astype(jnp.int32) + offs).reshape(N_EBLKS, 1, EBLK)
    s_oh = jax.nn.one_hot(S.reshape(N), VOCAB, dtype=f32)
    mask_col = mask.reshape(N, 1)

    row = lambda v: v.reshape(1, -1)
    layers = ['enc%d' % i for i in range(N_ENC)] + \
             ['dec%d' % i for i in range(N_DEC)]
    stk = lambda name: jnp.stack([p[pre + name] for pre in layers])
    stkr = lambda name: jnp.stack([row(p[pre + name]) for pre in layers])

    he = pl.pallas_call(
        _eemb_body,
        grid=(8,),
        in_specs=[pl.BlockSpec((E // 8, EDGE_IN), lambda i: (i, 0)),
                  _full((EDGE_IN, H)), _full((1, H)),
                  _full((1, H)), _full((1, H))],
        out_specs=pl.BlockSpec((E // 8, H), lambda i: (i, 0)),
        out_shape=jax.ShapeDtypeStruct((E, H), bf16),
        compiler_params=pltpu.CompilerParams(
            dimension_semantics=("parallel",)),
    )(e_raw, p['edge_W'], row(p['edge_b']),
      row(p['edge_ln_g']), row(p['edge_ln_b']))

    logits, prjs = pl.pallas_call(
        _mega_body,
        grid=(),
        in_specs=[_full((N, NODE_IN)), _full((N, VOCAB)), _full((N, 1)),
                  _full((N_EBLKS, 1, EBLK)),
                  pl.BlockSpec(memory_space=pl.ANY),
                  _full((NODE_IN, H)), _full((1, H)), _full((VOCAB, H)),
                  _full((1, H)), _full((1, H)),
                  _full((NLAYERS, H, H)), _full((NLAYERS, H, H)),
                  _full((NLAYERS, H, H)), _full((NLAYERS, 1, H)),
                  _full((NLAYERS, H, H)), _full((NLAYERS, 1, H)),
                  _full((NLAYERS, H, H)), _full((NLAYERS, 1, H)),
                  _full((NLAYERS, 1, H)), _full((NLAYERS, 1, H)),
                  _full((NLAYERS, H, 4 * H)), _full((NLAYERS, 1, 4 * H)),
                  _full((NLAYERS, 4 * H, H)), _full((NLAYERS, 1, H)),
                  _full((NLAYERS, 1, H)), _full((NLAYERS, 1, H)),
                  _full((NLAYERS, H, H)), _full((NLAYERS, 1, H)),
                  _full((NLAYERS, 1, H)), _full((NLAYERS, 1, H)),
                  _full((H, VOCAB)), _full((1, VOCAB)),
                  _full((H, H)), _full((H, H)), _full((1, H))],
        out_specs=[_full((N, VOCAB)), _full((B, H))],
        out_shape=[jax.ShapeDtypeStruct((N, VOCAB), f32),
                   jax.ShapeDtypeStruct((B, H), f32)],
        scratch_shapes=[pltpu.VMEM((N, H), f32), pltpu.VMEM((N, H), f32),
                        pltpu.VMEM((N, H), f32), pltpu.VMEM((N, H), f32),
                        pltpu.VMEM((2, EBLK, H), bf16),
                        pltpu.SemaphoreType.DMA((2,))],
    )(node_raw, s_oh, mask_col, src, he,
      p['node_W'], row(p['node_b']), p['W_s'],
      row(p['node_ln_g']), row(p['node_ln_b']),
      stk('_m1_W')[:, :H], stk('_m1_W')[:, H:2 * H], stk('_m1_W')[:, 2 * H:],
      stkr('_m1_b'),
      stk('_m2_W'), stkr('_m2_b'), stk('_m3_W'), stkr('_m3_b'),
      stkr('_ln1_g'), stkr('_ln1_b'),
      stk('_f1_W'), stkr('_f1_b'), stk('_f2_W'), stkr('_f2_b'),
      stkr('_ln2_g'), stkr('_ln2_b'),
      stk('_skip_W'), stkr('_skip_b'), stkr('_skln_g'), stkr('_skln_b'),
      p['readout_W'], row(p['readout_b']),
      p['proj1_W'], p['proj2_W'], row(p['proj2_b']))

    return logits, S.reshape(-1), prjs


# speedup vs baseline: 1.3250x; 1.3129x over previous
import functools

import jax
import jax.numpy as jnp
import numpy as np
from jax import lax
from jax.experimental import pallas as pl
from jax.experimental.pallas import tpu as pltpu

B, L, H, K, VOCAB = 4, 512, 256, 16, 4
N_ENC, N_DEC = 3, 3
N_RBF, N_POS = 16, 16
NODE_IN = 9
EDGE_IN = N_RBF + N_POS

N = B * L
E = B * L * K
NLAYERS = N_ENC + N_DEC
EBLK = 2048
NBLK = EBLK // K
N_EBLKS = E // EBLK
BLKS_PER_BATCH = (L * K) // EBLK
f32 = jnp.float32
bf16 = jnp.bfloat16


def _ln(x, g, b):
    mu = jnp.mean(x, -1, keepdims=True)
    var = jnp.var(x, -1, keepdims=True)
    return (x - mu) / jnp.sqrt(var + 1e-5) * g + b


def _knn_body(c_ref, ct_ref, keys_ref, a_s):
    c = c_ref[0]
    ct = ct_ref[0]
    g = jnp.dot(c, ct, preferred_element_type=f32)
    n2c = jnp.sum(c * c, axis=1, keepdims=True)
    n2r = jnp.sum(ct * ct, axis=0, keepdims=True)
    d2 = jnp.maximum(n2c + n2r - 2.0 * g, 0.0)
    d = jnp.sqrt(d2 + 1e-8)
    ri = lax.broadcasted_iota(jnp.int32, (L, L), 0)
    ci = lax.broadcasted_iota(jnp.int32, (L, L), 1)
    bits = lax.bitcast_convert_type(d, jnp.int32)
    key = (bits & jnp.int32(~511)) | ri
    key = jnp.where(ri == ci, jnp.int32(0x7F000000), key)
    a_s[...] = key

    def ext(k, carry):
        a = a_s[...]
        x = jnp.minimum(a[:256], a[256:])
        x = jnp.minimum(x[:128], x[128:])
        x = jnp.minimum(x[:64], x[64:])
        x = jnp.minimum(x[:32], x[32:])
        x = jnp.minimum(x[:16], x[16:])
        x = jnp.minimum(x[:8], x[8:])
        mn = jnp.min(x, axis=0, keepdims=True)
        keys_ref[0, pl.ds(k, 1), :] = mn
        a_s[...] = jnp.where(ri == (mn & 511), jnp.int32(0x7F000000), a)
        return carry

    lax.fori_loop(0, K, ext, 0)


def _eemb_body(eraw_ref, ew_ref, eb_ref, g_ref, b_ref, he_ref):
    h = jnp.dot(eraw_ref[...], ew_ref[...], preferred_element_type=f32)
    he_ref[...] = _ln(h + eb_ref[...], g_ref[...], b_ref[...]).astype(bf16)


def _mega_body(nraw_ref, soh_ref, mask_ref, src_ref, he_hbm,
               nw_ref, nb_ref, ws_ref, nlg_ref, nlb_ref,
               w1e_ref, w1s_ref, w1d_ref, b1_ref,
               w2_ref, b2_ref, w3_ref, b3_ref,
               g1_ref, bb1_ref, f1_ref, fb1_ref, f2_ref, fb2_ref,
               g2_ref, bb2_ref, wsk_ref, bsk_ref, gsk_ref, bsk2_ref,
               wr_ref, br_ref, p1_ref, p2_ref, p2b_ref,
               logits_ref, prj_ref,
               hv_s, a_s, bd_s, agg_s, he_buf, sem):
    h = jnp.dot(nraw_ref[...], nw_ref[...], preferred_element_type=f32)
    h = h + nb_ref[...] + jnp.dot(soh_ref[...], ws_ref[...],
                                  preferred_element_type=f32)
    hv0 = _ln(h, nlg_ref[...], nlb_ref[...]) * mask_ref[...]
    hv_s[...] = hv0
    a_s[...] = jnp.dot(hv0, w1s_ref[0], preferred_element_type=f32)
    bd_s[...] = jnp.dot(hv0, w1d_ref[0], preferred_element_type=f32)

    def he_copy(blk, slot):
        return pltpu.make_async_copy(
            he_hbm.at[pl.ds(blk * EBLK, EBLK), :], he_buf.at[slot],
            sem.at[slot])

    def layer_body(l, carry):
        w1e_b = w1e_ref[l].astype(bf16)
        b1v = b1_ref[l]
        w2v = w2_ref[l].astype(bf16)
        b2v = b2_ref[l]

        he_copy(0, 0).start()

        def edge_body(blk, c):
            slot = lax.rem(blk, 2)
            he_copy(blk, slot).wait()

            @pl.when(blk + 1 < N_EBLKS)
            def _():
                he_copy(blk + 1, lax.rem(blk + 1, 2)).start()

            base = (blk // BLKS_PER_BATCH) * L
            src_local = (src_ref[blk, 0, :] - base).astype(jnp.int16)
            oh = jnp.where(
                src_local[:, None] ==
                lax.broadcasted_iota(jnp.int16, (EBLK, L), 1),
                bf16(1.0), bf16(0.0))
            a_blk = a_s[pl.ds(base, L), :].astype(bf16)
            gath = jnp.dot(oh, a_blk, preferred_element_type=f32)
            epart = jnp.dot(he_buf[slot], w1e_b,
                            preferred_element_type=f32)
            pre = (epart + b1v + gath).reshape(NBLK, K, H) \
                + bd_s[pl.ds(blk * NBLK, NBLK), :][:, None, :]
            m = jax.nn.gelu(pre.astype(bf16)).reshape(EBLK, H)
            m2 = jnp.dot(m, w2v, preferred_element_type=f32) + b2v
            m2 = jax.nn.gelu(m2.astype(bf16)).astype(f32)
            agg_s[pl.ds(blk * NBLK, NBLK), :] = \
                jnp.sum(m2.reshape(NBLK, K, H), axis=1)
            return c

        lax.fori_loop(0, N_EBLKS, edge_body, 0)

        fdot = lambda x, w: jnp.dot(x, w, preferred_element_type=f32)
        hv = hv_s[...]
        agg = fdot(agg_s[...] * (1.0 / K), w3_ref[l]) + b3_ref[l]
        hh = _ln(hv + agg, g1_ref[l], bb1_ref[l])
        ff = fdot(jnp.maximum(fdot(hh, f1_ref[l]) + fb1_ref[l], 0.0),
                  f2_ref[l]) + fb2_ref[l]
        hh = _ln(hh + ff, g2_ref[l], bb2_ref[l])
        sk = jnp.maximum(fdot(hh, wsk_ref[l]) + bsk_ref[l], 0.0)
        hv_new = hv + _ln(sk, gsk_ref[l], bsk2_ref[l])
        hv_s[...] = hv_new
        ln = jnp.minimum(l + 1, NLAYERS - 1)
        a_s[...] = jnp.dot(hv_new, w1s_ref[ln], preferred_element_type=f32)
        bd_s[...] = jnp.dot(hv_new, w1d_ref[ln], preferred_element_type=f32)
        return carry

    lax.fori_loop(0, NLAYERS, layer_body, 0)

    hv = hv_s[...]
    logits_ref[...] = jnp.dot(hv, wr_ref[...],
                              preferred_element_type=f32) + br_ref[...]
    ge = jnp.mean(hv.reshape(B, L, H), axis=1)
    prj = jnp.maximum(jnp.dot(ge, p1_ref[...],
                              preferred_element_type=f32), 0.0)
    prj_ref[...] = jnp.dot(prj, p2_ref[...],
                           preferred_element_type=f32) + p2b_ref[...]


def _full(shape):
    return pl.BlockSpec(shape, lambda *_: tuple(0 for _ in shape))


def kernel(X, S, mask, params):
    p = params

    center = X[:, :, 1, :]
    c0 = center - jnp.mean(center, axis=1, keepdims=True)
    c0t = jnp.swapaxes(c0, 1, 2)
    keys = pl.pallas_call(
        _knn_body,
        grid=(B,),
        in_specs=[pl.BlockSpec((1, L, 3), lambda i: (i, 0, 0)),
                  pl.BlockSpec((1, 3, L), lambda i: (i, 0, 0))],
        out_specs=pl.BlockSpec((1, K, L), lambda i: (i, 0, 0)),
        out_shape=jax.ShapeDtypeStruct((B, K, L), jnp.int32),
        scratch_shapes=[pltpu.VMEM((L, L), jnp.int32)],
        compiler_params=pltpu.CompilerParams(
            dimension_semantics=("arbitrary",)),
    )(c0, c0t)
    nbr = jnp.swapaxes(keys & 511, 1, 2)
    d_nbr = jnp.swapaxes(
        lax.bitcast_convert_type(keys & jnp.int32(~511), f32), 1, 2)
    centers = jnp.linspace(2.0, 22.0, N_RBF)
    sigma = (22.0 - 2.0) / N_RBF
    rbf = jnp.exp(-(((d_nbr[..., None] - centers) / sigma) ** 2))
    rel = (nbr - jnp.arange(L)[None, :, None]).astype(f32)
    freq = jnp.exp(-jnp.arange(N_POS // 2, dtype=f32)
                   * (np.log(10000.0) / (N_POS // 2)))
    ang = rel[..., None] * freq
    posenc = jnp.concatenate([jnp.sin(ang), jnp.cos(ang)], -1)
    e_raw = jnp.concatenate([rbf, posenc], -1).reshape(E, EDGE_IN)

    def unit(v):
        return v / (jnp.linalg.norm(v, axis=-1, keepdims=True) + 1e-8)
    v1 = unit(X[:, :, 1] - X[:, :, 0])
    v2 = unit(X[:, :, 2] - X[:, :, 1])
    v3 = unit(jnp.roll(center, -1, axis=1) - center)
    node_raw = jnp.concatenate([v1, v2, v3], -1).reshape(N, NODE_IN)

    offs = (jnp.arange(B, dtype=jnp.int32) * L)[:, None, None]
    src = (nbr.astype(jnp.int32) + offs).reshape(N_EBLKS, 1, EBLK)
    s_oh = jax.nn.one_hot(S.reshape(N), VOCAB, dtype=f32)
    mask_col = mask.reshape(N, 1)

    row = lambda v: v.reshape(1, -1)
    layers = ['enc%d' % i for i in range(N_ENC)] + \
             ['dec%d' % i for i in range(N_DEC)]
    stk = lambda name: jnp.stack([p[pre + name] for pre in layers])
    stkr = lambda name: jnp.stack([row(p[pre + name]) for pre in layers])

    he = pl.pallas_call(
        _eemb_body,
        grid=(8,),
        in_specs=[pl.BlockSpec((E // 8, EDGE_IN), lambda i: (i, 0)),
                  _full((EDGE_IN, H)), _full((1, H)),
                  _full((1, H)), _full((1, H))],
        out_specs=pl.BlockSpec((E // 8, H), lambda i: (i, 0)),
        out_shape=jax.ShapeDtypeStruct((E, H), bf16),
        compiler_params=pltpu.CompilerParams(
            dimension_semantics=("parallel",)),
    )(e_raw, p['edge_W'], row(p['edge_b']),
      row(p['edge_ln_g']), row(p['edge_ln_b']))

    logits, prjs = pl.pallas_call(
        _mega_body,
        grid=(),
        in_specs=[_full((N, NODE_IN)), _full((N, VOCAB)), _full((N, 1)),
                  _full((N_EBLKS, 1, EBLK)),
                  pl.BlockSpec(memory_space=pl.ANY),
                  _full((NODE_IN, H)), _full((1, H)), _full((VOCAB, H)),
                  _full((1, H)), _full((1, H)),
                  _full((NLAYERS, H, H)), _full((NLAYERS, H, H)),
                  _full((NLAYERS, H, H)), _full((NLAYERS, 1, H)),
                  _full((NLAYERS, H, H)), _full((NLAYERS, 1, H)),
                  _full((NLAYERS, H, H)), _full((NLAYERS, 1, H)),
                  _full((NLAYERS, 1, H)), _full((NLAYERS, 1, H)),
                  _full((NLAYERS, H, 4 * H)), _full((NLAYERS, 1, 4 * H)),
                  _full((NLAYERS, 4 * H, H)), _full((NLAYERS, 1, H)),
                  _full((NLAYERS, 1, H)), _full((NLAYERS, 1, H)),
                  _full((NLAYERS, H, H)), _full((NLAYERS, 1, H)),
                  _full((NLAYERS, 1, H)), _full((NLAYERS, 1, H)),
                  _full((H, VOCAB)), _full((1, VOCAB)),
                  _full((H, H)), _full((H, H)), _full((1, H))],
        out_specs=[_full((N, VOCAB)), _full((B, H))],
        out_shape=[jax.ShapeDtypeStruct((N, VOCAB), f32),
                   jax.ShapeDtypeStruct((B, H), f32)],
        scratch_shapes=[pltpu.VMEM((N, H), f32), pltpu.VMEM((N, H), f32),
                        pltpu.VMEM((N, H), f32), pltpu.VMEM((N, H), f32),
                        pltpu.VMEM((2, EBLK, H), bf16),
                        pltpu.SemaphoreType.DMA((2,))],
    )(node_raw, s_oh, mask_col, src, he,
      p['node_W'], row(p['node_b']), p['W_s'],
      row(p['node_ln_g']), row(p['node_ln_b']),
      stk('_m1_W')[:, :H], stk('_m1_W')[:, H:2 * H], stk('_m1_W')[:, 2 * H:],
      stkr('_m1_b'),
      stk('_m2_W'), stkr('_m2_b'), stk('_m3_W'), stkr('_m3_b'),
      stkr('_ln1_g'), stkr('_ln1_b'),
      stk('_f1_W'), stkr('_f1_b'), stk('_f2_W'), stkr('_f2_b'),
      stkr('_ln2_g'), stkr('_ln2_b'),
      stk('_skip_W'), stkr('_skip_b'), stkr('_skln_g'), stkr('_skln_b'),
      p['readout_W'], row(p['readout_b']),
      p['proj1_W'], p['proj2_W'], row(p['proj2_b']))

    return logits, S.reshape(-1), prjs


# speedup vs baseline: 1.3257x; 1.0005x over previous
import functools

import jax
import jax.numpy as jnp
import numpy as np
from jax import lax
from jax.experimental import pallas as pl
from jax.experimental.pallas import tpu as pltpu

B, L, H, K, VOCAB = 4, 512, 256, 16, 4
N_ENC, N_DEC = 3, 3
N_RBF, N_POS = 16, 16
NODE_IN = 9
EDGE_IN = N_RBF + N_POS

N = B * L
E = B * L * K
NLAYERS = N_ENC + N_DEC
EBLK = 2048
NBLK = EBLK // K
N_EBLKS = E // EBLK
BLKS_PER_BATCH = (L * K) // EBLK
f32 = jnp.float32
bf16 = jnp.bfloat16


def _ln(x, g, b):
    mu = jnp.mean(x, -1, keepdims=True)
    var = jnp.var(x, -1, keepdims=True)
    return (x - mu) / jnp.sqrt(var + 1e-5) * g + b


def _knn_body(c_ref, ct_ref, keys_ref, a_s):
    c = c_ref[0]
    ct = ct_ref[0]
    dx = c[:, 0:1] - ct[0:1, :]
    dy = c[:, 1:2] - ct[1:2, :]
    dz = c[:, 2:3] - ct[2:3, :]
    d2 = dx * dx + dy * dy + dz * dz
    d = jnp.sqrt(d2 + 1e-8)
    ri = lax.broadcasted_iota(jnp.int32, (L, L), 0)
    ci = lax.broadcasted_iota(jnp.int32, (L, L), 1)
    bits = lax.bitcast_convert_type(d, jnp.int32)
    key = (bits & jnp.int32(~511)) | ri
    key = jnp.where(ri == ci, jnp.int32(0x7F000000), key)
    a_s[...] = key

    def ext(k, carry):
        a = a_s[...]
        x = jnp.minimum(a[:256], a[256:])
        x = jnp.minimum(x[:128], x[128:])
        x = jnp.minimum(x[:64], x[64:])
        x = jnp.minimum(x[:32], x[32:])
        x = jnp.minimum(x[:16], x[16:])
        x = jnp.minimum(x[:8], x[8:])
        mn = jnp.min(x, axis=0, keepdims=True)
        keys_ref[0, pl.ds(k, 1), :] = mn
        a_s[...] = jnp.where(ri == (mn & 511), jnp.int32(0x7F000000), a)
        return carry

    lax.fori_loop(0, K, ext, 0)


def _eemb_body(eraw_ref, ew_ref, eb_ref, g_ref, b_ref, he_ref):
    h = jnp.dot(eraw_ref[...], ew_ref[...], preferred_element_type=f32)
    he_ref[...] = _ln(h + eb_ref[...], g_ref[...], b_ref[...]).astype(bf16)


def _mega_body(nraw_ref, soh_ref, mask_ref, src_ref, he_hbm,
               nw_ref, nb_ref, ws_ref, nlg_ref, nlb_ref,
               w1e_ref, w1s_ref, w1d_ref, b1_ref,
               w2_ref, b2_ref, w3_ref, b3_ref,
               g1_ref, bb1_ref, f1_ref, fb1_ref, f2_ref, fb2_ref,
               g2_ref, bb2_ref, wsk_ref, bsk_ref, gsk_ref, bsk2_ref,
               wr_ref, br_ref, p1_ref, p2_ref, p2b_ref,
               logits_ref, prj_ref,
               hv_s, a_s, bd_s, agg_s, he_buf, sem):
    h = jnp.dot(nraw_ref[...], nw_ref[...], preferred_element_type=f32)
    h = h + nb_ref[...] + jnp.dot(soh_ref[...], ws_ref[...],
                                  preferred_element_type=f32)
    hv0 = _ln(h, nlg_ref[...], nlb_ref[...]) * mask_ref[...]
    hv_s[...] = hv0
    a_s[...] = jnp.dot(hv0, w1s_ref[0], preferred_element_type=f32)
    bd_s[...] = jnp.dot(hv0, w1d_ref[0], preferred_element_type=f32)

    def he_copy(blk, slot):
        return pltpu.make_async_copy(
            he_hbm.at[pl.ds(blk * EBLK, EBLK), :], he_buf.at[slot],
            sem.at[slot])

    def layer_body(l, carry):
        w1e_b = w1e_ref[l].astype(bf16)
        b1v = b1_ref[l]
        w2v = w2_ref[l].astype(bf16)
        b2v = b2_ref[l]

        he_copy(0, 0).start()

        def edge_body(blk, c):
            slot = lax.rem(blk, 2)
            he_copy(blk, slot).wait()

            @pl.when(blk + 1 < N_EBLKS)
            def _():
                he_copy(blk + 1, lax.rem(blk + 1, 2)).start()

            base = (blk // BLKS_PER_BATCH) * L
            src_local = (src_ref[blk, 0, :] - base).astype(jnp.int16)
            oh = jnp.where(
                src_local[:, None] ==
                lax.broadcasted_iota(jnp.int16, (EBLK, L), 1),
                bf16(1.0), bf16(0.0))
            a_blk = a_s[pl.ds(base, L), :].astype(bf16)
            gath = jnp.dot(oh, a_blk, preferred_element_type=f32)
            epart = jnp.dot(he_buf[slot], w1e_b,
                            preferred_element_type=f32)
            pre = (epart + b1v + gath).reshape(NBLK, K, H) \
                + bd_s[pl.ds(blk * NBLK, NBLK), :][:, None, :]
            m = jax.nn.gelu(pre.astype(bf16)).reshape(EBLK, H)
            m2 = jnp.dot(m, w2v, preferred_element_type=f32) + b2v
            m2 = jax.nn.gelu(m2.astype(bf16)).astype(f32)
            agg_s[pl.ds(blk * NBLK, NBLK), :] = \
                jnp.sum(m2.reshape(NBLK, K, H), axis=1)
            return c

        lax.fori_loop(0, N_EBLKS, edge_body, 0)

        fdot = lambda x, w: jnp.dot(x, w, preferred_element_type=f32)
        hv = hv_s[...]
        agg = fdot(agg_s[...] * (1.0 / K), w3_ref[l]) + b3_ref[l]
        hh = _ln(hv + agg, g1_ref[l], bb1_ref[l])
        ff = fdot(jnp.maximum(fdot(hh, f1_ref[l]) + fb1_ref[l], 0.0),
                  f2_ref[l]) + fb2_ref[l]
        hh = _ln(hh + ff, g2_ref[l], bb2_ref[l])
        sk = jnp.maximum(fdot(hh, wsk_ref[l]) + bsk_ref[l], 0.0)
        hv_new = hv + _ln(sk, gsk_ref[l], bsk2_ref[l])
        hv_s[...] = hv_new
        ln = jnp.minimum(l + 1, NLAYERS - 1)
        a_s[...] = jnp.dot(hv_new, w1s_ref[ln], preferred_element_type=f32)
        bd_s[...] = jnp.dot(hv_new, w1d_ref[ln], preferred_element_type=f32)
        return carry

    lax.fori_loop(0, NLAYERS, layer_body, 0)

    hv = hv_s[...]
    logits_ref[...] = jnp.dot(hv, wr_ref[...],
                              preferred_element_type=f32) + br_ref[...]
    ge = jnp.mean(hv.reshape(B, L, H), axis=1)
    prj = jnp.maximum(jnp.dot(ge, p1_ref[...],
                              preferred_element_type=f32), 0.0)
    prj_ref[...] = jnp.dot(prj, p2_ref[...],
                           preferred_element_type=f32) + p2b_ref[...]


def _full(shape):
    return pl.BlockSpec(shape, lambda *_: tuple(0 for _ in shape))


def kernel(X, S, mask, params):
    p = params

    center = X[:, :, 1, :]
    c0 = center
    c0t = jnp.swapaxes(c0, 1, 2)
    keys = pl.pallas_call(
        _knn_body,
        grid=(B,),
        in_specs=[pl.BlockSpec((1, L, 3), lambda i: (i, 0, 0)),
                  pl.BlockSpec((1, 3, L), lambda i: (i, 0, 0))],
        out_specs=pl.BlockSpec((1, K, L), lambda i: (i, 0, 0)),
        out_shape=jax.ShapeDtypeStruct((B, K, L), jnp.int32),
        scratch_shapes=[pltpu.VMEM((L, L), jnp.int32)],
        compiler_params=pltpu.CompilerParams(
            dimension_semantics=("arbitrary",)),
    )(c0, c0t)
    nbr = jnp.swapaxes(keys & 511, 1, 2)
    d_nbr = jnp.swapaxes(
        lax.bitcast_convert_type(keys & jnp.int32(~511), f32), 1, 2)
    centers = jnp.linspace(2.0, 22.0, N_RBF)
    sigma = (22.0 - 2.0) / N_RBF
    rbf = jnp.exp(-(((d_nbr[..., None] - centers) / sigma) ** 2))
    rel = (nbr - jnp.arange(L)[None, :, None]).astype(f32)
    freq = jnp.exp(-jnp.arange(N_POS // 2, dtype=f32)
                   * (np.log(10000.0) / (N_POS // 2)))
    ang = rel[..., None] * freq
    posenc = jnp.concatenate([jnp.sin(ang), jnp.cos(ang)], -1)
    e_raw = jnp.concatenate([rbf, posenc], -1).reshape(E, EDGE_IN)

    def unit(v):
        return v / (jnp.linalg.norm(v, axis=-1, keepdims=True) + 1e-8)
    v1 = unit(X[:, :, 1] - X[:, :, 0])
    v2 = unit(X[:, :, 2] - X[:, :, 1])
    v3 = unit(jnp.roll(center, -1, axis=1) - center)
    node_raw = jnp.concatenate([v1, v2, v3], -1).reshape(N, NODE_IN)

    offs = (jnp.arange(B, dtype=jnp.int32) * L)[:, None, None]
    src = (nbr.astype(jnp.int32) + offs).reshape(N_EBLKS, 1, EBLK)
    s_oh = jax.nn.one_hot(S.reshape(N), VOCAB, dtype=f32)
    mask_col = mask.reshape(N, 1)

    row = lambda v: v.reshape(1, -1)
    layers = ['enc%d' % i for i in range(N_ENC)] + \
             ['dec%d' % i for i in range(N_DEC)]
    stk = lambda name: jnp.stack([p[pre + name] for pre in layers])
    stkr = lambda name: jnp.stack([row(p[pre + name]) for pre in layers])

    he = pl.pallas_call(
        _eemb_body,
        grid=(8,),
        in_specs=[pl.BlockSpec((E // 8, EDGE_IN), lambda i: (i, 0)),
                  _full((EDGE_IN, H)), _full((1, H)),
                  _full((1, H)), _full((1, H))],
        out_specs=pl.BlockSpec((E // 8, H), lambda i: (i, 0)),
        out_shape=jax.ShapeDtypeStruct((E, H), bf16),
        compiler_params=pltpu.CompilerParams(
            dimension_semantics=("parallel",)),
    )(e_raw, p['edge_W'], row(p['edge_b']),
      row(p['edge_ln_g']), row(p['edge_ln_b']))

    logits, prjs = pl.pallas_call(
        _mega_body,
        grid=(),
        in_specs=[_full((N, NODE_IN)), _full((N, VOCAB)), _full((N, 1)),
                  _full((N_EBLKS, 1, EBLK)),
                  pl.BlockSpec(memory_space=pl.ANY),
                  _full((NODE_IN, H)), _full((1, H)), _full((VOCAB, H)),
                  _full((1, H)), _full((1, H)),
                  _full((NLAYERS, H, H)), _full((NLAYERS, H, H)),
                  _full((NLAYERS, H, H)), _full((NLAYERS, 1, H)),
                  _full((NLAYERS, H, H)), _full((NLAYERS, 1, H)),
                  _full((NLAYERS, H, H)), _full((NLAYERS, 1, H)),
                  _full((NLAYERS, 1, H)), _full((NLAYERS, 1, H)),
                  _full((NLAYERS, H, 4 * H)), _full((NLAYERS, 1, 4 * H)),
                  _full((NLAYERS, 4 * H, H)), _full((NLAYERS, 1, H)),
                  _full((NLAYERS, 1, H)), _full((NLAYERS, 1, H)),
                  _full((NLAYERS, H, H)), _full((NLAYERS, 1, H)),
                  _full((NLAYERS, 1, H)), _full((NLAYERS, 1, H)),
                  _full((H, VOCAB)), _full((1, VOCAB)),
                  _full((H, H)), _full((H, H)), _full((1, H))],
        out_specs=[_full((N, VOCAB)), _full((B, H))],
        out_shape=[jax.ShapeDtypeStruct((N, VOCAB), f32),
                   jax.ShapeDtypeStruct((B, H), f32)],
        scratch_shapes=[pltpu.VMEM((N, H), f32), pltpu.VMEM((N, H), f32),
                        pltpu.VMEM((N, H), f32), pltpu.VMEM((N, H), f32),
                        pltpu.VMEM((2, EBLK, H), bf16),
                        pltpu.SemaphoreType.DMA((2,))],
    )(node_raw, s_oh, mask_col, src, he,
      p['node_W'], row(p['node_b']), p['W_s'],
      row(p['node_ln_g']), row(p['node_ln_b']),
      stk('_m1_W')[:, :H], stk('_m1_W')[:, H:2 * H], stk('_m1_W')[:, 2 * H:],
      stkr('_m1_b'),
      stk('_m2_W'), stkr('_m2_b'), stk('_m3_W'), stkr('_m3_b'),
      stkr('_ln1_g'), stkr('_ln1_b'),
      stk('_f1_W'), stkr('_f1_b'), stk('_f2_W'), stkr('_f2_b'),
      stkr('_ln2_g'), stkr('_ln2_b'),
      stk('_skip_W'), stkr('_skip_b'), stkr('_skln_g'), stkr('_skln_b'),
      p['readout_W'], row(p['readout_b']),
      p['proj1_W'], p['proj2_W'], row(p['proj2_b']))

    return logits, S.reshape(-1), prjs


# speedup vs baseline: 1.3897x; 1.0483x over previous
import functools

import jax
import jax.numpy as jnp
import numpy as np
from jax import lax
from jax.experimental import pallas as pl
from jax.experimental.pallas import tpu as pltpu

B, L, H, K, VOCAB = 4, 512, 256, 16, 4
N_ENC, N_DEC = 3, 3
N_RBF, N_POS = 16, 16
NODE_IN = 9
EDGE_IN = N_RBF + N_POS

N = B * L
E = B * L * K
NLAYERS = N_ENC + N_DEC
EBLK = 4096
NBLK = EBLK // K
N_EBLKS = E // EBLK
BLKS_PER_BATCH = (L * K) // EBLK
f32 = jnp.float32
bf16 = jnp.bfloat16


def _ln(x, g, b):
    mu = jnp.mean(x, -1, keepdims=True)
    var = jnp.var(x, -1, keepdims=True)
    return (x - mu) / jnp.sqrt(var + 1e-5) * g + b


def _knn_body(c_ref, ct_ref, keys_ref, a_s):
    c = c_ref[0]
    ct = ct_ref[0]
    dx = c[:, 0:1] - ct[0:1, :]
    dy = c[:, 1:2] - ct[1:2, :]
    dz = c[:, 2:3] - ct[2:3, :]
    d2 = dx * dx + dy * dy + dz * dz
    d = jnp.sqrt(d2 + 1e-8)
    ri = lax.broadcasted_iota(jnp.int32, (L, L), 0)
    ci = lax.broadcasted_iota(jnp.int32, (L, L), 1)
    bits = lax.bitcast_convert_type(d, jnp.int32)
    key = (bits & jnp.int32(~511)) | ri
    key = jnp.where(ri == ci, jnp.int32(0x7F000000), key)
    a_s[...] = key

    def ext(k, carry):
        a = a_s[...]
        x = jnp.minimum(a[:256], a[256:])
        x = jnp.minimum(x[:128], x[128:])
        x = jnp.minimum(x[:64], x[64:])
        x = jnp.minimum(x[:32], x[32:])
        x = jnp.minimum(x[:16], x[16:])
        x = jnp.minimum(x[:8], x[8:])
        mn = jnp.min(x, axis=0, keepdims=True)
        keys_ref[0, pl.ds(k, 1), :] = mn
        a_s[...] = jnp.where(ri == (mn & 511), jnp.int32(0x7F000000), a)
        return carry

    lax.fori_loop(0, K, ext, 0)


def _eemb_body(eraw_ref, ew_ref, eb_ref, g_ref, b_ref, he_ref):
    h = jnp.dot(eraw_ref[...], ew_ref[...], preferred_element_type=f32)
    he_ref[...] = _ln(h + eb_ref[...], g_ref[...], b_ref[...]).astype(bf16)


def _mega_body(nraw_ref, soh_ref, mask_ref, src_ref, he_hbm,
               nw_ref, nb_ref, ws_ref, nlg_ref, nlb_ref,
               w1e_ref, w1s_ref, w1d_ref, b1_ref,
               w2_ref, b2_ref, w3_ref, b3_ref,
               g1_ref, bb1_ref, f1_ref, fb1_ref, f2_ref, fb2_ref,
               g2_ref, bb2_ref, wsk_ref, bsk_ref, gsk_ref, bsk2_ref,
               wr_ref, br_ref, p1_ref, p2_ref, p2b_ref,
               logits_ref, prj_ref,
               hv_s, a_s, bd_s, agg_s, he_buf, sem):
    h = jnp.dot(nraw_ref[...], nw_ref[...], preferred_element_type=f32)
    h = h + nb_ref[...] + jnp.dot(soh_ref[...], ws_ref[...],
                                  preferred_element_type=f32)
    hv0 = _ln(h, nlg_ref[...], nlb_ref[...]) * mask_ref[...]
    hv_s[...] = hv0
    a_s[...] = jnp.dot(hv0, w1s_ref[0], preferred_element_type=f32)
    bd_s[...] = jnp.dot(hv0, w1d_ref[0], preferred_element_type=f32)

    def he_copy(blk, slot):
        return pltpu.make_async_copy(
            he_hbm.at[pl.ds(blk * EBLK, EBLK), :], he_buf.at[slot],
            sem.at[slot])

    def layer_body(l, carry):
        w1e_b = w1e_ref[l].astype(bf16)
        b1v = b1_ref[l]
        w2v = w2_ref[l].astype(bf16)
        b2v = b2_ref[l]

        he_copy(0, 0).start()

        def edge_body(blk, c):
            slot = lax.rem(blk, 2)
            he_copy(blk, slot).wait()

            @pl.when(blk + 1 < N_EBLKS)
            def _():
                he_copy(blk + 1, lax.rem(blk + 1, 2)).start()

            base = (blk // BLKS_PER_BATCH) * L
            src_local = (src_ref[blk, 0, :] - base).astype(jnp.int16)
            oh = jnp.where(
                src_local[:, None] ==
                lax.broadcasted_iota(jnp.int16, (EBLK, L), 1),
                bf16(1.0), bf16(0.0))
            a_blk = a_s[pl.ds(base, L), :].astype(bf16)
            gath = jnp.dot(oh, a_blk, preferred_element_type=f32)
            epart = jnp.dot(he_buf[slot], w1e_b,
                            preferred_element_type=f32)
            pre = (epart + b1v + gath).reshape(NBLK, K, H) \
                + bd_s[pl.ds(blk * NBLK, NBLK), :][:, None, :]
            m = jax.nn.gelu(pre.astype(bf16)).reshape(EBLK, H)
            m2 = jnp.dot(m, w2v, preferred_element_type=f32) + b2v
            m2 = jax.nn.gelu(m2.astype(bf16)).astype(f32)
            agg_s[pl.ds(blk * NBLK, NBLK), :] = \
                jnp.sum(m2.reshape(NBLK, K, H), axis=1)
            return c

        lax.fori_loop(0, N_EBLKS, edge_body, 0)

        fdot = lambda x, w: jnp.dot(x, w, preferred_element_type=f32)
        hv = hv_s[...]
        agg = fdot(agg_s[...] * (1.0 / K), w3_ref[l]) + b3_ref[l]
        hh = _ln(hv + agg, g1_ref[l], bb1_ref[l])
        ff = fdot(jnp.maximum(fdot(hh, f1_ref[l]) + fb1_ref[l], 0.0),
                  f2_ref[l]) + fb2_ref[l]
        hh = _ln(hh + ff, g2_ref[l], bb2_ref[l])
        sk = jnp.maximum(fdot(hh, wsk_ref[l]) + bsk_ref[l], 0.0)
        hv_new = hv + _ln(sk, gsk_ref[l], bsk2_ref[l])
        hv_s[...] = hv_new
        ln = jnp.minimum(l + 1, NLAYERS - 1)
        a_s[...] = jnp.dot(hv_new, w1s_ref[ln], preferred_element_type=f32)
        bd_s[...] = jnp.dot(hv_new, w1d_ref[ln], preferred_element_type=f32)
        return carry

    lax.fori_loop(0, NLAYERS, layer_body, 0)

    hv = hv_s[...]
    logits_ref[...] = jnp.dot(hv, wr_ref[...],
                              preferred_element_type=f32) + br_ref[...]
    ge = jnp.mean(hv.reshape(B, L, H), axis=1)
    prj = jnp.maximum(jnp.dot(ge, p1_ref[...],
                              preferred_element_type=f32), 0.0)
    prj_ref[...] = jnp.dot(prj, p2_ref[...],
                           preferred_element_type=f32) + p2b_ref[...]


def _full(shape):
    return pl.BlockSpec(shape, lambda *_: tuple(0 for _ in shape))


def kernel(X, S, mask, params):
    p = params

    center = X[:, :, 1, :]
    c0 = center
    c0t = jnp.swapaxes(c0, 1, 2)
    keys = pl.pallas_call(
        _knn_body,
        grid=(B,),
        in_specs=[pl.BlockSpec((1, L, 3), lambda i: (i, 0, 0)),
                  pl.BlockSpec((1, 3, L), lambda i: (i, 0, 0))],
        out_specs=pl.BlockSpec((1, K, L), lambda i: (i, 0, 0)),
        out_shape=jax.ShapeDtypeStruct((B, K, L), jnp.int32),
        scratch_shapes=[pltpu.VMEM((L, L), jnp.int32)],
        compiler_params=pltpu.CompilerParams(
            dimension_semantics=("arbitrary",)),
    )(c0, c0t)
    nbr = jnp.swapaxes(keys & 511, 1, 2)
    d_nbr = jnp.swapaxes(
        lax.bitcast_convert_type(keys & jnp.int32(~511), f32), 1, 2)
    centers = jnp.linspace(2.0, 22.0, N_RBF)
    sigma = (22.0 - 2.0) / N_RBF
    rbf = jnp.exp(-(((d_nbr[..., None] - centers) / sigma) ** 2))
    rel = (nbr - jnp.arange(L)[None, :, None]).astype(f32)
    freq = jnp.exp(-jnp.arange(N_POS // 2, dtype=f32)
                   * (np.log(10000.0) / (N_POS // 2)))
    ang = rel[..., None] * freq
    posenc = jnp.concatenate([jnp.sin(ang), jnp.cos(ang)], -1)
    e_raw = jnp.concatenate([rbf, posenc], -1).reshape(E, EDGE_IN)

    def unit(v):
        return v / (jnp.linalg.norm(v, axis=-1, keepdims=True) + 1e-8)
    v1 = unit(X[:, :, 1] - X[:, :, 0])
    v2 = unit(X[:, :, 2] - X[:, :, 1])
    v3 = unit(jnp.roll(center, -1, axis=1) - center)
    node_raw = jnp.concatenate([v1, v2, v3], -1).reshape(N, NODE_IN)

    offs = (jnp.arange(B, dtype=jnp.int32) * L)[:, None, None]
    src = (nbr.astype(jnp.int32) + offs).reshape(N_EBLKS, 1, EBLK)
    s_oh = jax.nn.one_hot(S.reshape(N), VOCAB, dtype=f32)
    mask_col = mask.reshape(N, 1)

    row = lambda v: v.reshape(1, -1)
    layers = ['enc%d' % i for i in range(N_ENC)] + \
             ['dec%d' % i for i in range(N_DEC)]
    stk = lambda name: jnp.stack([p[pre + name] for pre in layers])
    stkr = lambda name: jnp.stack([row(p[pre + name]) for pre in layers])

    he = pl.pallas_call(
        _eemb_body,
        grid=(8,),
        in_specs=[pl.BlockSpec((E // 8, EDGE_IN), lambda i: (i, 0)),
                  _full((EDGE_IN, H)), _full((1, H)),
                  _full((1, H)), _full((1, H))],
        out_specs=pl.BlockSpec((E // 8, H), lambda i: (i, 0)),
        out_shape=jax.ShapeDtypeStruct((E, H), bf16),
        compiler_params=pltpu.CompilerParams(
            dimension_semantics=("parallel",)),
    )(e_raw, p['edge_W'], row(p['edge_b']),
      row(p['edge_ln_g']), row(p['edge_ln_b']))

    logits, prjs = pl.pallas_call(
        _mega_body,
        grid=(),
        in_specs=[_full((N, NODE_IN)), _full((N, VOCAB)), _full((N, 1)),
                  _full((N_EBLKS, 1, EBLK)),
                  pl.BlockSpec(memory_space=pl.ANY),
                  _full((NODE_IN, H)), _full((1, H)), _full((VOCAB, H)),
                  _full((1, H)), _full((1, H)),
                  _full((NLAYERS, H, H)), _full((NLAYERS, H, H)),
                  _full((NLAYERS, H, H)), _full((NLAYERS, 1, H)),
                  _full((NLAYERS, H, H)), _full((NLAYERS, 1, H)),
                  _full((NLAYERS, H, H)), _full((NLAYERS, 1, H)),
                  _full((NLAYERS, 1, H)), _full((NLAYERS, 1, H)),
                  _full((NLAYERS, H, 4 * H)), _full((NLAYERS, 1, 4 * H)),
                  _full((NLAYERS, 4 * H, H)), _full((NLAYERS, 1, H)),
                  _full((NLAYERS, 1, H)), _full((NLAYERS, 1, H)),
                  _full((NLAYERS, H, H)), _full((NLAYERS, 1, H)),
                  _full((NLAYERS, 1, H)), _full((NLAYERS, 1, H)),
                  _full((H, VOCAB)), _full((1, VOCAB)),
                  _full((H, H)), _full((H, H)), _full((1, H))],
        out_specs=[_full((N, VOCAB)), _full((B, H))],
        out_shape=[jax.ShapeDtypeStruct((N, VOCAB), f32),
                   jax.ShapeDtypeStruct((B, H), f32)],
        scratch_shapes=[pltpu.VMEM((N, H), f32), pltpu.VMEM((N, H), f32),
                        pltpu.VMEM((N, H), f32), pltpu.VMEM((N, H), f32),
                        pltpu.VMEM((2, EBLK, H), bf16),
                        pltpu.SemaphoreType.DMA((2,))],
    )(node_raw, s_oh, mask_col, src, he,
      p['node_W'], row(p['node_b']), p['W_s'],
      row(p['node_ln_g']), row(p['node_ln_b']),
      stk('_m1_W')[:, :H], stk('_m1_W')[:, H:2 * H], stk('_m1_W')[:, 2 * H:],
      stkr('_m1_b'),
      stk('_m2_W'), stkr('_m2_b'), stk('_m3_W'), stkr('_m3_b'),
      stkr('_ln1_g'), stkr('_ln1_b'),
      stk('_f1_W'), stkr('_f1_b'), stk('_f2_W'), stkr('_f2_b'),
      stkr('_ln2_g'), stkr('_ln2_b'),
      stk('_skip_W'), stkr('_skip_b'), stkr('_skln_g'), stkr('_skln_b'),
      p['readout_W'], row(p['readout_b']),
      p['proj1_W'], p['proj2_W'], row(p['proj2_b']))

    return logits, S.reshape(-1), prjs


# speedup vs baseline: 1.3912x; 1.0011x over previous
import jax
import jax.numpy as jnp
import numpy as np
from jax import lax
from jax.experimental import pallas as pl
from jax.experimental.pallas import tpu as pltpu

B, L, H, K, VOCAB = 4, 512, 256, 16, 4
N_ENC, N_DEC = 3, 3
N_RBF, N_POS = 16, 16
NODE_IN = 9
EDGE_IN = N_RBF + N_POS

N = B * L
E = B * L * K
NLAYERS = N_ENC + N_DEC
EBLK = 4096
NBLK = EBLK // K
N_EBLKS = E // EBLK
BLKS_PER_BATCH = (L * K) // EBLK
f32 = jnp.float32
bf16 = jnp.bfloat16


def _ln(x, g, b):
    mu = jnp.mean(x, -1, keepdims=True)
    var = jnp.var(x, -1, keepdims=True)
    return (x - mu) / jnp.sqrt(var + 1e-5) * g + b


def _knn_body(c_ref, ct_ref, keys_ref, a_s):
    c = c_ref[0]
    ct = ct_ref[0]
    dx = c[:, 0:1] - ct[0:1, :]
    dy = c[:, 1:2] - ct[1:2, :]
    dz = c[:, 2:3] - ct[2:3, :]
    d2 = dx * dx + dy * dy + dz * dz
    d = jnp.sqrt(d2 + 1e-8)
    ri = lax.broadcasted_iota(jnp.int32, (L, L), 0)
    ci = lax.broadcasted_iota(jnp.int32, (L, L), 1)
    bits = lax.bitcast_convert_type(d, jnp.int32)
    key = (bits & jnp.int32(~511)) | ri
    key = jnp.where(ri == ci, jnp.int32(0x7F000000), key)
    a_s[...] = key

    def ext(k, carry):
        a = a_s[...]
        x = jnp.minimum(a[:256], a[256:])
        x = jnp.minimum(x[:128], x[128:])
        x = jnp.minimum(x[:64], x[64:])
        x = jnp.minimum(x[:32], x[32:])
        x = jnp.minimum(x[:16], x[16:])
        x = jnp.minimum(x[:8], x[8:])
        mn = jnp.min(x, axis=0, keepdims=True)
        keys_ref[0, pl.ds(k, 1), :] = mn
        a_s[...] = jnp.where(ri == (mn & 511), jnp.int32(0x7F000000), a)
        return carry

    lax.fori_loop(0, K, ext, 0)


def _eemb_body(eraw_ref, ew_ref, eb_ref, g_ref, b_ref, he_ref):
    h = jnp.dot(eraw_ref[...], ew_ref[...], preferred_element_type=f32)
    he_ref[...] = _ln(h + eb_ref[...], g_ref[...], b_ref[...]).astype(bf16)


def _mega_body(nraw_ref, soh_ref, mask_ref, src_ref, he_hbm,
               nw_ref, nb_ref, ws_ref, nlg_ref, nlb_ref,
               w1e_ref, w1s_ref, w1d_ref, b1_ref,
               w2_ref, b2_ref, w3_ref, b3_ref,
               g1_ref, bb1_ref, f1_ref, fb1_ref, f2_ref, fb2_ref,
               g2_ref, bb2_ref, wsk_ref, bsk_ref, gsk_ref, bsk2_ref,
               wr_ref, br_ref, p1_ref, p2_ref, p2b_ref,
               logits_ref, prj_ref,
               hv_s, a_s, bd_s, agg_s, he_buf, sem):
    h = jnp.dot(nraw_ref[...], nw_ref[...], preferred_element_type=f32)
    h = h + nb_ref[...] + jnp.dot(soh_ref[...], ws_ref[...],
                                  preferred_element_type=f32)
    hv0 = _ln(h, nlg_ref[...], nlb_ref[...]) * mask_ref[...]
    hv_s[...] = hv0
    a_s[...] = jnp.dot(hv0, w1s_ref[0], preferred_element_type=f32)
    bd_s[...] = jnp.dot(hv0, w1d_ref[0], preferred_element_type=f32)

    def he_copy(blk, slot):
        return pltpu.make_async_copy(
            he_hbm.at[pl.ds(blk * EBLK, EBLK), :], he_buf.at[slot],
            sem.at[slot])

    def layer_body(l, carry):
        w1e_b = w1e_ref[l].astype(bf16)
        b1v = b1_ref[l]
        w2v = w2_ref[l].astype(bf16)
        b2v = b2_ref[l]

        he_copy(0, 0).start()

        def edge_body(blk, c):
            slot = lax.rem(blk, 2)
            he_copy(blk, slot).wait()

            @pl.when(blk + 1 < N_EBLKS)
            def _():
                he_copy(blk + 1, lax.rem(blk + 1, 2)).start()

            base = (blk // BLKS_PER_BATCH) * L
            src_local = (src_ref[blk, 0, :] - base).astype(jnp.int16)
            oh = jnp.where(
                src_local[:, None] ==
                lax.broadcasted_iota(jnp.int16, (EBLK, L), 1),
                bf16(1.0), bf16(0.0))
            a_blk = a_s[pl.ds(base, L), :].astype(bf16)
            gath = jnp.dot(oh, a_blk, preferred_element_type=f32)
            epart = jnp.dot(he_buf[slot], w1e_b,
                            preferred_element_type=f32)
            pre = (epart + b1v + gath).reshape(NBLK, K, H) \
                + bd_s[pl.ds(blk * NBLK, NBLK), :][:, None, :]
            m = jax.nn.gelu(pre.astype(bf16)).reshape(EBLK, H)
            m2 = jnp.dot(m, w2v, preferred_element_type=f32) + b2v
            m2 = jax.nn.gelu(m2.astype(bf16)).astype(f32)
            agg_s[pl.ds(blk * NBLK, NBLK), :] = \
                jnp.sum(m2.reshape(NBLK, K, H), axis=1)
            return c

        lax.fori_loop(0, N_EBLKS, edge_body, 0)

        fdot = lambda x, w: jnp.dot(x, w, preferred_element_type=f32)
        hv = hv_s[...]
        agg = fdot(agg_s[...] * (1.0 / K), w3_ref[l]) + b3_ref[l]
        hh = _ln(hv + agg, g1_ref[l], bb1_ref[l])
        ff = fdot(jnp.maximum(fdot(hh, f1_ref[l]) + fb1_ref[l], 0.0),
                  f2_ref[l]) + fb2_ref[l]
        hh = _ln(hh + ff, g2_ref[l], bb2_ref[l])
        sk = jnp.maximum(fdot(hh, wsk_ref[l]) + bsk_ref[l], 0.0)
        hv_new = hv + _ln(sk, gsk_ref[l], bsk2_ref[l])
        hv_s[...] = hv_new
        ln = jnp.minimum(l + 1, NLAYERS - 1)
        a_s[...] = jnp.dot(hv_new, w1s_ref[ln], preferred_element_type=f32)
        bd_s[...] = jnp.dot(hv_new, w1d_ref[ln], preferred_element_type=f32)
        return carry

    lax.fori_loop(0, NLAYERS, layer_body, 0)

    hv = hv_s[...]
    logits_ref[...] = jnp.dot(hv, wr_ref[...],
                              preferred_element_type=f32) + br_ref[...]
    ge = jnp.mean(hv.reshape(B, L, H), axis=1)
    prj = jnp.maximum(jnp.dot(ge, p1_ref[...],
                              preferred_element_type=f32), 0.0)
    prj_ref[...] = jnp.dot(prj, p2_ref[...],
                           preferred_element_type=f32) + p2b_ref[...]


def _full(shape):
    return pl.BlockSpec(shape, lambda *_: tuple(0 for _ in shape))


def kernel(X, S, mask, params):
    p = params

    center = X[:, :, 1, :]
    c0 = center
    c0t = jnp.swapaxes(c0, 1, 2)
    keys = pl.pallas_call(
        _knn_body,
        grid=(B,),
        in_specs=[pl.BlockSpec((1, L, 3), lambda i: (i, 0, 0)),
                  pl.BlockSpec((1, 3, L), lambda i: (i, 0, 0))],
        out_specs=pl.BlockSpec((1, K, L), lambda i: (i, 0, 0)),
        out_shape=jax.ShapeDtypeStruct((B, K, L), jnp.int32),
        scratch_shapes=[pltpu.VMEM((L, L), jnp.int32)],
        compiler_params=pltpu.CompilerParams(
            dimension_semantics=("arbitrary",)),
    )(c0, c0t)
    nbr = jnp.swapaxes(keys & 511, 1, 2)
    d_nbr = jnp.swapaxes(
        lax.bitcast_convert_type(keys & jnp.int32(~511), f32), 1, 2)
    centers = jnp.linspace(2.0, 22.0, N_RBF)
    sigma = (22.0 - 2.0) / N_RBF
    rbf = jnp.exp(-(((d_nbr[..., None] - centers) / sigma) ** 2))
    rel = (nbr - jnp.arange(L)[None, :, None]).astype(f32)
    freq = jnp.exp(-jnp.arange(N_POS // 2, dtype=f32)
                   * (np.log(10000.0) / (N_POS // 2)))
    ang = rel[..., None] * freq
    posenc = jnp.concatenate([jnp.sin(ang), jnp.cos(ang)], -1)
    e_raw = jnp.concatenate([rbf, posenc], -1).reshape(E, EDGE_IN)

    def unit(v):
        return v / (jnp.linalg.norm(v, axis=-1, keepdims=True) + 1e-8)
    v1 = unit(X[:, :, 1] - X[:, :, 0])
    v2 = unit(X[:, :, 2] - X[:, :, 1])
    v3 = unit(jnp.roll(center, -1, axis=1) - center)
    node_raw = jnp.concatenate([v1, v2, v3], -1).reshape(N, NODE_IN)

    offs = (jnp.arange(B, dtype=jnp.int32) * L)[:, None, None]
    src = (nbr.astype(jnp.int32) + offs).reshape(N_EBLKS, 1, EBLK)
    s_oh = jax.nn.one_hot(S.reshape(N), VOCAB, dtype=f32)
    mask_col = mask.reshape(N, 1)

    row = lambda v: v.reshape(1, -1)
    layers = ['enc%d' % i for i in range(N_ENC)] + \
             ['dec%d' % i for i in range(N_DEC)]
    stk = lambda name: jnp.stack([p[pre + name] for pre in layers])
    stkr = lambda name: jnp.stack([row(p[pre + name]) for pre in layers])

    he = pl.pallas_call(
        _eemb_body,
        grid=(8,),
        in_specs=[pl.BlockSpec((E // 8, EDGE_IN), lambda i: (i, 0)),
                  _full((EDGE_IN, H)), _full((1, H)),
                  _full((1, H)), _full((1, H))],
        out_specs=pl.BlockSpec((E // 8, H), lambda i: (i, 0)),
        out_shape=jax.ShapeDtypeStruct((E, H), bf16),
        compiler_params=pltpu.CompilerParams(
            dimension_semantics=("parallel",)),
    )(e_raw, p['edge_W'], row(p['edge_b']),
      row(p['edge_ln_g']), row(p['edge_ln_b']))

    logits, prjs = pl.pallas_call(
        _mega_body,
        grid=(),
        in_specs=[_full((N, NODE_IN)), _full((N, VOCAB)), _full((N, 1)),
                  _full((N_EBLKS, 1, EBLK)),
                  pl.BlockSpec(memory_space=pl.ANY),
                  _full((NODE_IN, H)), _full((1, H)), _full((VOCAB, H)),
                  _full((1, H)), _full((1, H)),
                  _full((NLAYERS, H, H)), _full((NLAYERS, H, H)),
                  _full((NLAYERS, H, H)), _full((NLAYERS, 1, H)),
                  _full((NLAYERS, H, H)), _full((NLAYERS, 1, H)),
                  _full((NLAYERS, H, H)), _full((NLAYERS, 1, H)),
                  _full((NLAYERS, 1, H)), _full((NLAYERS, 1, H)),
                  _full((NLAYERS, H, 4 * H)), _full((NLAYERS, 1, 4 * H)),
                  _full((NLAYERS, 4 * H, H)), _full((NLAYERS, 1, H)),
                  _full((NLAYERS, 1, H)), _full((NLAYERS, 1, H)),
                  _full((NLAYERS, H, H)), _full((NLAYERS, 1, H)),
                  _full((NLAYERS, 1, H)), _full((NLAYERS, 1, H)),
                  _full((H, VOCAB)), _full((1, VOCAB)),
                  _full((H, H)), _full((H, H)), _full((1, H))],
        out_specs=[_full((N, VOCAB)), _full((B, H))],
        out_shape=[jax.ShapeDtypeStruct((N, VOCAB), f32),
                   jax.ShapeDtypeStruct((B, H), f32)],
        scratch_shapes=[pltpu.VMEM((N, H), f32), pltpu.VMEM((N, H), f32),
                        pltpu.VMEM((N, H), f32), pltpu.VMEM((N, H), f32),
                        pltpu.VMEM((2, EBLK, H), bf16),
                        pltpu.SemaphoreType.DMA((2,))],
    )(node_raw, s_oh, mask_col, src, he,
      p['node_W'], row(p['node_b']), p['W_s'],
      row(p['node_ln_g']), row(p['node_ln_b']),
      stk('_m1_W')[:, :H], stk('_m1_W')[:, H:2 * H], stk('_m1_W')[:, 2 * H:],
      stkr('_m1_b'),
      stk('_m2_W'), stkr('_m2_b'), stk('_m3_W'), stkr('_m3_b'),
      stkr('_ln1_g'), stkr('_ln1_b'),
      stk('_f1_W'), stkr('_f1_b'), stk('_f2_W'), stkr('_f2_b'),
      stkr('_ln2_g'), stkr('_ln2_b'),
      stk('_skip_W'), stkr('_skip_b'), stkr('_skln_g'), stkr('_skln_b'),
      p['readout_W'], row(p['readout_b']),
      p['proj1_W'], p['proj2_W'], row(p['proj2_b']))

    return logits, S.reshape(-1), prjs
